# baseline (speedup 1.0000x reference)
import functools

import jax
import jax.numpy as jnp
from jax import lax
from jax.experimental import pallas as pl
from jax.experimental.pallas import tpu as pltpu
from jax.experimental.pallas import tpu_sc as plsc

_NC = 2
_NS = 16
_NW = _NC * _NS
_B = 96
_H = 128


@functools.partial(jax.jit, static_argnames=("n", "epw"))
def _spmm_half(h, col2, row3, rowt2, zrows, *, n, epw):
    ch = epw // _B
    tl = epw - ch * _B
    nb = 3
    rpw = (n // _NS) // 8 * 8
    rlast = n - rpw * (_NS - 1)
    mesh = plsc.VectorSubcoreMesh(core_axis_name="c", subcore_axis_name="s")

    def body(h_hbm, col_hbm, row_hbm, rowt_hbm, z_hbm, out_hbm, colv, rowv,
             rowtv, b0, b1, b2, acc, g0, g1, g2, s0, s1, s2):
        bufs = [b0, b1, b2]
        gsems = [g0, g1, g2]
        ssems = [s0, s1, s2]
        c = lax.axis_index("c")
        s = lax.axis_index("s")
        w = c * _NS + s

        @pl.when(s < _NS - 1)
        def _():
            pltpu.sync_copy(z_hbm.at[pl.ds(0, rpw)], acc.at[pl.ds(s * rpw, rpw)])

        @pl.when(s == _NS - 1)
        def _():
            pltpu.sync_copy(z_hbm, acc.at[pl.ds(s * rpw, rlast)])

        pltpu.sync_copy(col_hbm.at[w], colv)
        pltpu.sync_copy(row_hbm.at[w], rowv)
        if tl:
            pltpu.sync_copy(rowt_hbm.at[w], rowtv)
        plsc.subcore_barrier()

        def pair(i0):
            gds = [pltpu.async_copy(
                h_hbm.at[colv.at[pl.ds((i0 + b) * _B, _B)]], bufs[b], gsems[b])
                for b in range(nb)]
            sds = []
            for b in range(nb):
                gds[b].wait()
                sds.append(pltpu.async_copy(bufs[b], acc.at[rowv.at[i0 + b]],
                                            ssems[b], add=True))
            for d in sds:
                d.wait()

        def step(j, carry):
            pair(j * nb)
            return carry

        lax.fori_loop(0, ch // nb, step, 0)
        for i in range(ch - ch % nb, ch):
            pltpu.async_copy(h_hbm.at[colv.at[pl.ds(i * _B, _B)]], bufs[0],
                             gsems[0]).wait()
            pltpu.async_copy(bufs[0], acc.at[rowv.at[i]], ssems[0],
                             add=True).wait()
        if tl:
            pltpu.async_copy(h_hbm.at[colv.at[pl.ds(ch * _B, tl)]],
                             bufs[0].at[pl.ds(0, tl)], gsems[0]).wait()
            pltpu.async_copy(bufs[0].at[pl.ds(0, tl)], acc.at[rowtv],
                             ssems[0], add=True).wait()
        plsc.subcore_barrier()

        @pl.when(s < _NS - 1)
        def _():
            pltpu.sync_copy(acc.at[pl.ds(s * rpw, rpw)],
                            out_hbm.at[c, pl.ds(s * rpw, rpw)])

        @pl.when(s == _NS - 1)
        def _():
            pltpu.sync_copy(acc.at[pl.ds(s * rpw, rlast)],
                            out_hbm.at[c, pl.ds(s * rpw, rlast)])

    return pl.kernel(
        body,
        out_type=jax.ShapeDtypeStruct((_NC, n, _H), jnp.float32),
        mesh=mesh,
        scratch_types=[
            pltpu.VMEM((epw,), jnp.int32),
            pltpu.VMEM((ch, _B), jnp.int32),
            pltpu.VMEM((max(tl, 8),), jnp.int32),
            pltpu.VMEM((_B, _H), jnp.float32),
            pltpu.VMEM((_B, _H), jnp.float32),
            pltpu.VMEM((_B, _H), jnp.float32),
            pltpu.VMEM_SHARED((n, _H), jnp.float32),
        ] + [pltpu.SemaphoreType.DMA for _ in range(6)],
    )(h, col2, row3, rowt2, zrows)


def _enc_in(x, w2, sen, n, rm):
    nr = n // rm
    d = x.shape[1]

    def body(sen_ref, x_ref, w_ref, oa_ref, ox_ref, s_ref):
        ids = lax.broadcasted_iota(jnp.int32, (1, d), 1)
        hit = ids == sen_ref[0]
        xv = x_ref[...]
        xm = jnp.where(hit, 0.0, xv)
        oa_ref[...] = jnp.dot(xm, w_ref[0], preferred_element_type=jnp.float32)
        ox_ref[...] = jnp.dot(xm, w_ref[1], preferred_element_type=jnp.float32)
        s_ref[...] = jnp.sum(jnp.where(hit, xv, 0.0), axis=1, keepdims=True)

    return pl.pallas_call(
        body,
        grid=(2, nr),
        in_specs=[
            pl.BlockSpec(memory_space=pltpu.SMEM),
            pl.BlockSpec((rm, d), lambda k, r: (r, 0)),
            pl.BlockSpec((2, d, _H), lambda k, r: (0, 0, 0)),
        ],
        out_specs=[
            pl.BlockSpec((rm, _H), lambda k, r: (k * nr + r, 0)),
            pl.BlockSpec((rm, _H), lambda k, r: (k * nr + r, 0)),
            pl.BlockSpec((rm, 1), lambda k, r: (r, 0)),
        ],
        out_shape=[
            jax.ShapeDtypeStruct((2 * n, _H), jnp.float32),
            jax.ShapeDtypeStruct((2 * n, _H), jnp.float32),
            jax.ShapeDtypeStruct((n, 1), jnp.float32),
        ],
    )(sen, x, w2)


def _enc_mid(sa_p, w, n, rm):
    nr = n // rm

    def body(x_ref, w_ref, o_ref):
        h = jnp.maximum(x_ref[0] + x_ref[1], 0.0)
        o_ref[...] = jnp.dot(h, w_ref[...], preferred_element_type=jnp.float32)

    return pl.pallas_call(
        body,
        grid=(2, nr),
        in_specs=[
            pl.BlockSpec((2, rm, _H), lambda k, r: (0, r, 0)),
            pl.BlockSpec((_H, _H), lambda k, r: (0, 0)),
        ],
        out_specs=pl.BlockSpec((rm, _H), lambda k, r: (k * nr + r, 0)),
        out_shape=jax.ShapeDtypeStruct((2 * n, _H), jnp.float32),
    )(sa_p, w)


def _decoder_a(za_p, s_col, w0, w1, b, n, rm, cn):
    nr = n // rm
    nc = pl.cdiv(n, cn)

    def body(z_ref, s_ref, w0_ref, w1_ref, b_ref, o_ref):
        z = z_ref[0] + z_ref[1]
        acc = jnp.dot(z, w0_ref[...], preferred_element_type=jnp.float32)
        o_ref[...] = jax.nn.sigmoid(acc + s_ref[...] * w1_ref[...] + b_ref[...])

    return pl.pallas_call(
        body,
        grid=(nr, nc),
        in_specs=[
            pl.BlockSpec((2, rm, _H), lambda r, c: (0, r, 0)),
            pl.BlockSpec((rm, 1), lambda r, c: (r, 0)),
            pl.BlockSpec((_H, cn), lambda r, c: (0, c)),
            pl.BlockSpec((1, cn), lambda r, c: (0, c)),
            pl.BlockSpec((1, cn), lambda r, c: (0, c)),
        ],
        out_specs=pl.BlockSpec((rm, cn), lambda r, c: (r, c)),
        out_shape=jax.ShapeDtypeStruct((n, n), jnp.float32),
    )(za_p, s_col, w0, w1, b)


def _decoder_xs(za_p, zx_p, s_col, wx0, wx1, bx, wsa, wsx, bsp, n, rm):
    nr = n // rm
    d = wx0.shape[1]

    def body(za_ref, zx_ref, s_ref, wx0_ref, wx1_ref, bx_ref,
             wsa_ref, wsx_ref, bsp_ref, xp_ref, sg_ref):
        za = za_ref[0] + za_ref[1]
        zx = zx_ref[0] + zx_ref[1]
        xp_ref[...] = (jnp.dot(zx, wx0_ref[...], preferred_element_type=jnp.float32)
                       + s_ref[...] * wx1_ref[...] + bx_ref[...])
        logits = (jnp.dot(za, wsa_ref[...], preferred_element_type=jnp.float32)
                  + jnp.dot(zx, wsx_ref[...], preferred_element_type=jnp.float32)
                  + bsp_ref[...])
        m = jnp.max(logits, axis=1, keepdims=True)
        ex = jnp.exp(logits - m)
        sg_ref[...] = ex / jnp.sum(ex, axis=1, keepdims=True)

    return pl.pallas_call(
        body,
        grid=(nr,),
        in_specs=[
            pl.BlockSpec((2, rm, _H), lambda r: (0, r, 0)),
            pl.BlockSpec((2, rm, _H), lambda r: (0, r, 0)),
            pl.BlockSpec((rm, 1), lambda r: (r, 0)),
            pl.BlockSpec((_H, d), lambda r: (0, 0)),
            pl.BlockSpec((1, d), lambda r: (0, 0)),
            pl.BlockSpec((1, d), lambda r: (0, 0)),
            pl.BlockSpec((_H, _H), lambda r: (0, 0)),
            pl.BlockSpec((_H, _H), lambda r: (0, 0)),
            pl.BlockSpec((1, _H), lambda r: (0, 0)),
        ],
        out_specs=[
            pl.BlockSpec((rm, d), lambda r: (r, 0)),
            pl.BlockSpec((rm, _H), lambda r: (r, 0)),
        ],
        out_shape=[
            jax.ShapeDtypeStruct((n, d), jnp.float32),
            jax.ShapeDtypeStruct((n, _H), jnp.float32),
        ],
    )(za_p, zx_p, s_col, wx0, wx1, bx, wsa, wsx, bsp)


def kernel(X, W_base_a, W_mean_a, W_logstd_a, W_base_x, W_mean_x, W_logstd_x,
           Wa, ba, Wx, bx, Ws, bs, edge_index, sen_idx):
    n, d = X.shape
    e = edge_index.shape[1]
    rm = 1000

    sen = jnp.asarray(sen_idx, dtype=jnp.int32).reshape(1)

    row = edge_index[0]
    col = edge_index[1]
    epw = e // _NW
    chf = epw // _B
    tl = epw - chf * _B
    col2 = col.reshape(_NW, epw) + jnp.where(
        jnp.arange(_NW, dtype=jnp.int32)[:, None] >= _NS, n, 0)
    roww = row.reshape(_NW, epw)
    row3 = roww[:, :chf * _B].reshape(_NW, chf, _B)
    rowt2 = roww[:, chf * _B:]
    rlast = n - (n // _NS) // 8 * 8 * (_NS - 1)
    zrows = jnp.zeros((rlast, _H), jnp.float32)

    wb2 = jnp.stack([W_base_a, W_base_x])
    xwad, xwxd, S = _enc_in(X, wb2, sen, n, 2000)
    sa_a = _spmm_half(xwad, col2, row3, rowt2, zrows, n=n, epw=epw)
    hwd_a = _enc_mid(sa_a, W_mean_a, n, 2000)
    za_p = _spmm_half(hwd_a, col2, row3, rowt2, zrows, n=n, epw=epw)
    sa_x = _spmm_half(xwxd, col2, row3, rowt2, zrows, n=n, epw=epw)
    hwd_x = _enc_mid(sa_x, W_mean_x, n, 2000)
    zx_p = _spmm_half(hwd_x, col2, row3, rowt2, zrows, n=n, epw=epw)

    A_pred = _decoder_a(za_p, S, Wa, ba.reshape(1, n), n, 2000, 2048)
    S2 = S + A_pred[:1, :1] * 0.0
    ws_pad = jnp.pad(Ws, ((0, 0), (0, _H - Ws.shape[1])))
    bs_pad = jnp.concatenate([bs, jnp.full((_H - bs.shape[0],), -1e30,
                                           jnp.float32)]).reshape(1, _H)
    X_pred, sg = _decoder_xs(za_p, zx_p, S2, Wx[:_H], Wx[_H:].reshape(1, d),
                             bx.reshape(1, d), ws_pad[:_H], ws_pad[_H:],
                             bs_pad, n, 2000)
    S_agg_pred = sg[:, :Ws.shape[1]]
    return (A_pred, X_pred, S_agg_pred)


# speedup vs baseline: 1.0653x; 1.0653x over previous
import functools

import jax
import jax.numpy as jnp
from jax import lax
from jax.experimental import pallas as pl
from jax.experimental.pallas import tpu as pltpu
from jax.experimental.pallas import tpu_sc as plsc

_NC = 2
_NS = 16
_NW = _NC * _NS
_B = 128
_H = 128


@functools.partial(jax.jit, static_argnames=("n", "epw"))
def _spmm_half(h, col2, row3, rowt2, zrows, *, n, epw):
    ch = epw // _B
    tl = epw - ch * _B
    nb = 2
    rpw = (n // _NS) // 8 * 8
    rlast = n - rpw * (_NS - 1)
    mesh = plsc.VectorSubcoreMesh(core_axis_name="c", subcore_axis_name="s")

    def body(h_hbm, col_hbm, row_hbm, rowt_hbm, z_hbm, out_hbm, colv, rowv,
             rowtv, b0, b1, acc, g0, g1, s0, s1):
        bufs = [b0, b1]
        gsems = [g0, g1]
        ssems = [s0, s1]
        c = lax.axis_index("c")
        s = lax.axis_index("s")
        w = c * _NS + s

        @pl.when(s < _NS - 1)
        def _():
            pltpu.sync_copy(z_hbm.at[pl.ds(0, rpw)], acc.at[pl.ds(s * rpw, rpw)])

        @pl.when(s == _NS - 1)
        def _():
            pltpu.sync_copy(z_hbm, acc.at[pl.ds(s * rpw, rlast)])

        pltpu.sync_copy(col_hbm.at[w], colv)
        pltpu.sync_copy(row_hbm.at[w], rowv)
        if tl:
            pltpu.sync_copy(rowt_hbm.at[w], rowtv)
        plsc.subcore_barrier()

        def pair(i0):
            gds = [pltpu.async_copy(
                h_hbm.at[colv.at[pl.ds((i0 + b) * _B, _B)]], bufs[b], gsems[b])
                for b in range(nb)]
            sds = []
            for b in range(nb):
                gds[b].wait()
                sds.append(pltpu.async_copy(bufs[b], acc.at[rowv.at[i0 + b]],
                                            ssems[b], add=True))
            for d in sds:
                d.wait()

        def step(j, carry):
            pair(j * nb)
            return carry

        lax.fori_loop(0, ch // nb, step, 0)
        for i in range(ch - ch % nb, ch):
            pltpu.async_copy(h_hbm.at[colv.at[pl.ds(i * _B, _B)]], bufs[0],
                             gsems[0]).wait()
            pltpu.async_copy(bufs[0], acc.at[rowv.at[i]], ssems[0],
                             add=True).wait()
        if tl:
            pltpu.async_copy(h_hbm.at[colv.at[pl.ds(ch * _B, tl)]],
                             bufs[0].at[pl.ds(0, tl)], gsems[0]).wait()
            pltpu.async_copy(bufs[0].at[pl.ds(0, tl)], acc.at[rowtv],
                             ssems[0], add=True).wait()
        plsc.subcore_barrier()

        @pl.when(s < _NS - 1)
        def _():
            pltpu.sync_copy(acc.at[pl.ds(s * rpw, rpw)],
                            out_hbm.at[c, pl.ds(s * rpw, rpw)])

        @pl.when(s == _NS - 1)
        def _():
            pltpu.sync_copy(acc.at[pl.ds(s * rpw, rlast)],
                            out_hbm.at[c, pl.ds(s * rpw, rlast)])

    return pl.kernel(
        body,
        out_type=jax.ShapeDtypeStruct((_NC, n, _H), jnp.float32),
        mesh=mesh,
        scratch_types=[
            pltpu.VMEM((epw,), jnp.int32),
            pltpu.VMEM((ch, _B), jnp.int32),
            pltpu.VMEM((max(tl, 8),), jnp.int32),
            pltpu.VMEM((_B, _H), jnp.float32),
            pltpu.VMEM((_B, _H), jnp.float32),
            pltpu.VMEM_SHARED((n, _H), jnp.float32),
        ] + [pltpu.SemaphoreType.DMA for _ in range(4)],
    )(h, col2, row3, rowt2, zrows)


def _enc_in(x, w2, sen, n, rm):
    nr = n // rm
    d = x.shape[1]

    def body(sen_ref, x_ref, w_ref, oa_ref, ox_ref, s_ref):
        ids = lax.broadcasted_iota(jnp.int32, (1, d), 1)
        hit = ids == sen_ref[0]
        xv = x_ref[...]
        xm = jnp.where(hit, 0.0, xv)
        oa_ref[...] = jnp.dot(xm, w_ref[0], preferred_element_type=jnp.float32)
        ox_ref[...] = jnp.dot(xm, w_ref[1], preferred_element_type=jnp.float32)
        s_ref[...] = jnp.sum(jnp.where(hit, xv, 0.0), axis=1, keepdims=True)

    return pl.pallas_call(
        body,
        grid=(2, nr),
        in_specs=[
            pl.BlockSpec(memory_space=pltpu.SMEM),
            pl.BlockSpec((rm, d), lambda k, r: (r, 0)),
            pl.BlockSpec((2, d, _H), lambda k, r: (0, 0, 0)),
        ],
        out_specs=[
            pl.BlockSpec((rm, _H), lambda k, r: (k * nr + r, 0)),
            pl.BlockSpec((rm, _H), lambda k, r: (k * nr + r, 0)),
            pl.BlockSpec((rm, 1), lambda k, r: (r, 0)),
        ],
        out_shape=[
            jax.ShapeDtypeStruct((2 * n, _H), jnp.float32),
            jax.ShapeDtypeStruct((2 * n, _H), jnp.float32),
            jax.ShapeDtypeStruct((n, 1), jnp.float32),
        ],
    )(sen, x, w2)


def _enc_mid(sa_p, w, n, rm):
    nr = n // rm

    def body(x_ref, w_ref, o_ref):
        h = jnp.maximum(x_ref[0] + x_ref[1], 0.0)
        o_ref[...] = jnp.dot(h, w_ref[...], preferred_element_type=jnp.float32)

    return pl.pallas_call(
        body,
        grid=(2, nr),
        in_specs=[
            pl.BlockSpec((2, rm, _H), lambda k, r: (0, r, 0)),
            pl.BlockSpec((_H, _H), lambda k, r: (0, 0)),
        ],
        out_specs=pl.BlockSpec((rm, _H), lambda k, r: (k * nr + r, 0)),
        out_shape=jax.ShapeDtypeStruct((2 * n, _H), jnp.float32),
    )(sa_p, w)


def _decoder_a(za_p, s_col, w0, w1, b, n, rm, cn):
    nr = n // rm
    nc = pl.cdiv(n, cn)

    def body(z_ref, s_ref, w0_ref, w1_ref, b_ref, o_ref):
        z = z_ref[0] + z_ref[1]
        acc = jnp.dot(z, w0_ref[...], preferred_element_type=jnp.float32)
        o_ref[...] = jax.nn.sigmoid(acc + s_ref[...] * w1_ref[...] + b_ref[...])

    return pl.pallas_call(
        body,
        grid=(nr, nc),
        in_specs=[
            pl.BlockSpec((2, rm, _H), lambda r, c: (0, r, 0)),
            pl.BlockSpec((rm, 1), lambda r, c: (r, 0)),
            pl.BlockSpec((_H, cn), lambda r, c: (0, c)),
            pl.BlockSpec((1, cn), lambda r, c: (0, c)),
            pl.BlockSpec((1, cn), lambda r, c: (0, c)),
        ],
        out_specs=pl.BlockSpec((rm, cn), lambda r, c: (r, c)),
        out_shape=jax.ShapeDtypeStruct((n, n), jnp.float32),
    )(za_p, s_col, w0, w1, b)


def _decoder_xs(za_p, zx_p, s_col, wx0, wx1, bx, wsa, wsx, bsp, n, rm):
    nr = n // rm
    d = wx0.shape[1]

    def body(za_ref, zx_ref, s_ref, wx0_ref, wx1_ref, bx_ref,
             wsa_ref, wsx_ref, bsp_ref, xp_ref, sg_ref):
        za = za_ref[0] + za_ref[1]
        zx = zx_ref[0] + zx_ref[1]
        xp_ref[...] = (jnp.dot(zx, wx0_ref[...], preferred_element_type=jnp.float32)
                       + s_ref[...] * wx1_ref[...] + bx_ref[...])
        logits = (jnp.dot(za, wsa_ref[...], preferred_element_type=jnp.float32)
                  + jnp.dot(zx, wsx_ref[...], preferred_element_type=jnp.float32)
                  + bsp_ref[...])
        m = jnp.max(logits, axis=1, keepdims=True)
        ex = jnp.exp(logits - m)
        sg_ref[...] = ex / jnp.sum(ex, axis=1, keepdims=True)

    return pl.pallas_call(
        body,
        grid=(nr,),
        in_specs=[
            pl.BlockSpec((2, rm, _H), lambda r: (0, r, 0)),
            pl.BlockSpec((2, rm, _H), lambda r: (0, r, 0)),
            pl.BlockSpec((rm, 1), lambda r: (r, 0)),
            pl.BlockSpec((_H, d), lambda r: (0, 0)),
            pl.BlockSpec((1, d), lambda r: (0, 0)),
            pl.BlockSpec((1, d), lambda r: (0, 0)),
            pl.BlockSpec((_H, _H), lambda r: (0, 0)),
            pl.BlockSpec((_H, _H), lambda r: (0, 0)),
            pl.BlockSpec((1, _H), lambda r: (0, 0)),
        ],
        out_specs=[
            pl.BlockSpec((rm, d), lambda r: (r, 0)),
            pl.BlockSpec((rm, _H), lambda r: (r, 0)),
        ],
        out_shape=[
            jax.ShapeDtypeStruct((n, d), jnp.float32),
            jax.ShapeDtypeStruct((n, _H), jnp.float32),
        ],
    )(za_p, zx_p, s_col, wx0, wx1, bx, wsa, wsx, bsp)


def kernel(X, W_base_a, W_mean_a, W_logstd_a, W_base_x, W_mean_x, W_logstd_x,
           Wa, ba, Wx, bx, Ws, bs, edge_index, sen_idx):
    n, d = X.shape
    e = edge_index.shape[1]
    rm = 1000

    sen = jnp.asarray(sen_idx, dtype=jnp.int32).reshape(1)

    row = edge_index[0]
    col = edge_index[1]
    epw = e // _NW
    chf = epw // _B
    tl = epw - chf * _B
    col2 = col.reshape(_NW, epw) + jnp.where(
        jnp.arange(_NW, dtype=jnp.int32)[:, None] >= _NS, n, 0)
    roww = row.reshape(_NW, epw)
    row3 = roww[:, :chf * _B].reshape(_NW, chf, _B)
    rowt2 = roww[:, chf * _B:]
    rlast = n - (n // _NS) // 8 * 8 * (_NS - 1)
    zrows = jnp.zeros((rlast, _H), jnp.float32)

    wb2 = jnp.stack([W_base_a, W_base_x])
    xwad, xwxd, S = _enc_in(X, wb2, sen, n, 2000)
    sa_a = _spmm_half(xwad, col2, row3, rowt2, zrows, n=n, epw=epw)
    hwd_a = _enc_mid(sa_a, W_mean_a, n, 2000)
    za_p = _spmm_half(hwd_a, col2, row3, rowt2, zrows, n=n, epw=epw)
    sa_x = _spmm_half(xwxd, col2, row3, rowt2, zrows, n=n, epw=epw)
    hwd_x = _enc_mid(sa_x, W_mean_x, n, 2000)
    zx_p = _spmm_half(hwd_x, col2, row3, rowt2, zrows, n=n, epw=epw)

    A_pred = _decoder_a(za_p, S, Wa, ba.reshape(1, n), n, 2000, 2048)
    S2 = S + A_pred[:1, :1] * 0.0
    ws_pad = jnp.pad(Ws, ((0, 0), (0, _H - Ws.shape[1])))
    bs_pad = jnp.concatenate([bs, jnp.full((_H - bs.shape[0],), -1e30,
                                           jnp.float32)]).reshape(1, _H)
    X_pred, sg = _decoder_xs(za_p, zx_p, S2, Wx[:_H], Wx[_H:].reshape(1, d),
                             bx.reshape(1, d), ws_pad[:_H], ws_pad[_H:],
                             bs_pad, n, rm)
    S_agg_pred = sg[:, :Ws.shape[1]]
    return (A_pred, X_pred, S_agg_pred)


# speedup vs baseline: 1.0691x; 1.0036x over previous
import functools

import jax
import jax.numpy as jnp
from jax import lax
from jax.experimental import pallas as pl
from jax.experimental.pallas import tpu as pltpu
from jax.experimental.pallas import tpu_sc as plsc

_NC = 2
_NS = 16
_NW = _NC * _NS
_B = 128
_H = 128


@functools.partial(jax.jit, static_argnames=("n", "epw"))
def _spmm_half(h, col2, row3, rowt2, zrows, *, n, epw):
    ch = epw // _B
    tl = epw - ch * _B
    nb = 2
    rpw = (n // _NS) // 8 * 8
    rlast = n - rpw * (_NS - 1)
    mesh = plsc.VectorSubcoreMesh(core_axis_name="c", subcore_axis_name="s")

    def body(h_hbm, col_hbm, row_hbm, rowt_hbm, z_hbm, out_hbm, colv, rowv,
             rowtv, b0, b1, acc, g0, g1, s0, s1):
        bufs = [b0, b1]
        gsems = [g0, g1]
        ssems = [s0, s1]
        c = lax.axis_index("c")
        s = lax.axis_index("s")
        w = c * _NS + s

        @pl.when(s < _NS - 1)
        def _():
            pltpu.sync_copy(z_hbm.at[pl.ds(0, rpw)], acc.at[pl.ds(s * rpw, rpw)])

        @pl.when(s == _NS - 1)
        def _():
            pltpu.sync_copy(z_hbm, acc.at[pl.ds(s * rpw, rlast)])

        pltpu.sync_copy(col_hbm.at[w], colv)
        pltpu.sync_copy(row_hbm.at[w], rowv)
        if tl:
            pltpu.sync_copy(rowt_hbm.at[w], rowtv)
        plsc.subcore_barrier()

        def pair(i0):
            gds = [pltpu.async_copy(
                h_hbm.at[colv.at[pl.ds((i0 + b) * _B, _B)]], bufs[b], gsems[b])
                for b in range(nb)]
            sds = []
            for b in range(nb):
                gds[b].wait()
                sds.append(pltpu.async_copy(bufs[b], acc.at[rowv.at[i0 + b]],
                                            ssems[b], add=True))
            for d in sds:
                d.wait()

        def step(j, carry):
            pair(j * nb)
            return carry

        lax.fori_loop(0, ch // nb, step, 0)
        for i in range(ch - ch % nb, ch):
            pltpu.async_copy(h_hbm.at[colv.at[pl.ds(i * _B, _B)]], bufs[0],
                             gsems[0]).wait()
            pltpu.async_copy(bufs[0], acc.at[rowv.at[i]], ssems[0],
                             add=True).wait()
        if tl:
            pltpu.async_copy(h_hbm.at[colv.at[pl.ds(ch * _B, tl)]],
                             bufs[0].at[pl.ds(0, tl)], gsems[0]).wait()
            pltpu.async_copy(bufs[0].at[pl.ds(0, tl)], acc.at[rowtv],
                             ssems[0], add=True).wait()
        plsc.subcore_barrier()

        @pl.when(s < _NS - 1)
        def _():
            pltpu.sync_copy(acc.at[pl.ds(s * rpw, rpw)],
                            out_hbm.at[c, pl.ds(s * rpw, rpw)])

        @pl.when(s == _NS - 1)
        def _():
            pltpu.sync_copy(acc.at[pl.ds(s * rpw, rlast)],
                            out_hbm.at[c, pl.ds(s * rpw, rlast)])

    return pl.kernel(
        body,
        out_type=jax.ShapeDtypeStruct((_NC, n, _H), jnp.float32),
        mesh=mesh,
        scratch_types=[
            pltpu.VMEM((epw,), jnp.int32),
            pltpu.VMEM((ch, _B), jnp.int32),
            pltpu.VMEM((max(tl, 8),), jnp.int32),
            pltpu.VMEM((_B, _H), jnp.float32),
            pltpu.VMEM((_B, _H), jnp.float32),
            pltpu.VMEM_SHARED((n, _H), jnp.float32),
        ] + [pltpu.SemaphoreType.DMA for _ in range(4)],
    )(h, col2, row3, rowt2, zrows)


def _enc_in(x, w2, sen, n, rm):
    nr = n // rm
    d = x.shape[1]

    def body(sen_ref, x_ref, w_ref, oa_ref, ox_ref, s_ref):
        ids = lax.broadcasted_iota(jnp.int32, (1, d), 1)
        hit = ids == sen_ref[0]
        xv = x_ref[...]
        xm = jnp.where(hit, 0.0, xv)
        oa_ref[...] = jnp.dot(xm, w_ref[0], preferred_element_type=jnp.float32)
        ox_ref[...] = jnp.dot(xm, w_ref[1], preferred_element_type=jnp.float32)
        s_ref[...] = jnp.sum(jnp.where(hit, xv, 0.0), axis=1, keepdims=True)

    return pl.pallas_call(
        body,
        grid=(2, nr),
        in_specs=[
            pl.BlockSpec(memory_space=pltpu.SMEM),
            pl.BlockSpec((rm, d), lambda k, r: (r, 0)),
            pl.BlockSpec((2, d, _H), lambda k, r: (0, 0, 0)),
        ],
        out_specs=[
            pl.BlockSpec((rm, _H), lambda k, r: (k * nr + r, 0)),
            pl.BlockSpec((rm, _H), lambda k, r: (k * nr + r, 0)),
            pl.BlockSpec((rm, 1), lambda k, r: (r, 0)),
        ],
        out_shape=[
            jax.ShapeDtypeStruct((2 * n, _H), jnp.float32),
            jax.ShapeDtypeStruct((2 * n, _H), jnp.float32),
            jax.ShapeDtypeStruct((n, 1), jnp.float32),
        ],
    )(sen, x, w2)


def _enc_mid(sa_p, w, n, rm):
    nr = n // rm

    def body(x_ref, w_ref, o_ref):
        h = jnp.maximum(x_ref[0] + x_ref[1], 0.0)
        o_ref[...] = jnp.dot(h, w_ref[...], preferred_element_type=jnp.float32)

    return pl.pallas_call(
        body,
        grid=(2, nr),
        in_specs=[
            pl.BlockSpec((2, rm, _H), lambda k, r: (0, r, 0)),
            pl.BlockSpec((_H, _H), lambda k, r: (0, 0)),
        ],
        out_specs=pl.BlockSpec((rm, _H), lambda k, r: (k * nr + r, 0)),
        out_shape=jax.ShapeDtypeStruct((2 * n, _H), jnp.float32),
    )(sa_p, w)


def _decoder_a(za_p, s_col, w0, w1, b, n, rm, cn):
    nr = n // rm
    nc = pl.cdiv(n, cn)

    def body(z_ref, s_ref, w0_ref, w1_ref, b_ref, o_ref):
        z = z_ref[0] + z_ref[1]
        acc = jnp.dot(z, w0_ref[...], preferred_element_type=jnp.float32)
        o_ref[...] = jax.nn.sigmoid(acc + s_ref[...] * w1_ref[...] + b_ref[...])

    return pl.pallas_call(
        body,
        grid=(nr, nc),
        in_specs=[
            pl.BlockSpec((2, rm, _H), lambda r, c: (0, r, 0)),
            pl.BlockSpec((rm, 1), lambda r, c: (r, 0)),
            pl.BlockSpec((_H, cn), lambda r, c: (0, c)),
            pl.BlockSpec((1, cn), lambda r, c: (0, c)),
            pl.BlockSpec((1, cn), lambda r, c: (0, c)),
        ],
        out_specs=pl.BlockSpec((rm, cn), lambda r, c: (r, c)),
        out_shape=jax.ShapeDtypeStruct((n, n), jnp.float32),
    )(za_p, s_col, w0, w1, b)


def _decoder_xs(za_p, zx_p, s_col, wx0, wx1, bx, wsa, wsx, bsp, n, rm):
    nr = n // rm
    d = wx0.shape[1]

    def body(za_ref, zx_ref, s_ref, wx0_ref, wx1_ref, bx_ref,
             wsa_ref, wsx_ref, bsp_ref, xp_ref, sg_ref):
        za = za_ref[0] + za_ref[1]
        zx = zx_ref[0] + zx_ref[1]
        xp_ref[...] = (jnp.dot(zx, wx0_ref[...], preferred_element_type=jnp.float32)
                       + s_ref[...] * wx1_ref[...] + bx_ref[...])
        logits = (jnp.dot(za, wsa_ref[...], preferred_element_type=jnp.float32)
                  + jnp.dot(zx, wsx_ref[...], preferred_element_type=jnp.float32)
                  + bsp_ref[...])
        m = jnp.max(logits, axis=1, keepdims=True)
        ex = jnp.exp(logits - m)
        sg_ref[...] = ex / jnp.sum(ex, axis=1, keepdims=True)

    return pl.pallas_call(
        body,
        grid=(nr,),
        in_specs=[
            pl.BlockSpec((2, rm, _H), lambda r: (0, r, 0)),
            pl.BlockSpec((2, rm, _H), lambda r: (0, r, 0)),
            pl.BlockSpec((rm, 1), lambda r: (r, 0)),
            pl.BlockSpec((_H, d), lambda r: (0, 0)),
            pl.BlockSpec((1, d), lambda r: (0, 0)),
            pl.BlockSpec((1, d), lambda r: (0, 0)),
            pl.BlockSpec((_H, _H), lambda r: (0, 0)),
            pl.BlockSpec((_H, _H), lambda r: (0, 0)),
            pl.BlockSpec((1, _H), lambda r: (0, 0)),
        ],
        out_specs=[
            pl.BlockSpec((rm, d), lambda r: (r, 0)),
            pl.BlockSpec((rm, _H), lambda r: (r, 0)),
        ],
        out_shape=[
            jax.ShapeDtypeStruct((n, d), jnp.float32),
            jax.ShapeDtypeStruct((n, _H), jnp.float32),
        ],
    )(za_p, zx_p, s_col, wx0, wx1, bx, wsa, wsx, bsp)


def kernel(X, W_base_a, W_mean_a, W_logstd_a, W_base_x, W_mean_x, W_logstd_x,
           Wa, ba, Wx, bx, Ws, bs, edge_index, sen_idx):
    n, d = X.shape
    e = edge_index.shape[1]
    rm = 1000

    sen = jnp.asarray(sen_idx, dtype=jnp.int32).reshape(1)

    row = edge_index[0]
    col = edge_index[1]
    epw = e // _NW
    chf = epw // _B
    tl = epw - chf * _B
    col2 = col.reshape(_NW, epw) + jnp.where(
        jnp.arange(_NW, dtype=jnp.int32)[:, None] >= _NS, n, 0)
    roww = row.reshape(_NW, epw)
    row3 = roww[:, :chf * _B].reshape(_NW, chf, _B)
    rowt2 = roww[:, chf * _B:]
    rlast = n - (n // _NS) // 8 * 8 * (_NS - 1)
    zrows = jnp.zeros((rlast, _H), jnp.float32)

    wb2 = jnp.stack([W_base_a, W_base_x])
    xwad, xwxd, S = _enc_in(X, wb2, sen, n, 2000)
    sa_a = _spmm_half(xwad, col2, row3, rowt2, zrows, n=n, epw=epw)
    hwd_a = _enc_mid(sa_a, W_mean_a, n, 2000)
    za_p = _spmm_half(hwd_a, col2, row3, rowt2, zrows, n=n, epw=epw)
    sa_x = _spmm_half(xwxd, col2, row3, rowt2, zrows, n=n, epw=epw)
    hwd_x = _enc_mid(sa_x, W_mean_x, n, 2000)
    zx_p = _spmm_half(hwd_x, col2, row3, rowt2, zrows, n=n, epw=epw)

    A_pred = _decoder_a(za_p, S, Wa, ba.reshape(1, n), n, 2000, 2048)
    S2 = S + A_pred[:1, :1] * 0.0
    ws_pad = jnp.pad(Ws, ((0, 0), (0, _H - Ws.shape[1])))
    bs_pad = jnp.concatenate([bs, jnp.full((_H - bs.shape[0],), -1e30,
                                           jnp.float32)]).reshape(1, _H)
    X_pred, sg = _decoder_xs(za_p, zx_p, S2, Wx[:_H], Wx[_H:].reshape(1, d),
                             bx.reshape(1, d), ws_pad[:_H], ws_pad[_H:],
                             bs_pad, n, 2000)
    S_agg_pred = sg[:, :Ws.shape[1]]
    return (A_pred, X_pred, S_agg_pred)
